# baseline (speedup 1.0000x reference)
import functools

import jax
import jax.numpy as jnp
from jax import lax
from jax.experimental import pallas as pl
from jax.experimental.pallas import tpu as pltpu
from jax.experimental.pallas import tpu_sc as plsc

B = 4
M = 8192
D = 768
NC = 2
NS = 16
NW = NC * NS
POS_PER_W = M // NW
CH = 16
CHUNKS = POS_PER_W // CH
CHW = CH * D
VECS = CHW // 16


def _pos_add_body(x_hbm, pos_hbm, out_hbm, pos_v, x_v):
    wid = lax.axis_index("s") * NC + lax.axis_index("c")
    pos_base = wid * (POS_PER_W * D)

    def chunk_body(c, carry):
        pbase = pos_base + c * CHW
        pltpu.sync_copy(pos_hbm.at[pl.ds(pbase, CHW)], pos_v)

        def batch_body(b, carry):
            xbase = b * (M * D) + pbase
            pltpu.sync_copy(x_hbm.at[pl.ds(xbase, CHW)], x_v)

            def vec_body(k, carry):
                sl = pl.ds(k * 16, 16)
                x_v[sl] += pos_v[sl]
                return carry

            lax.fori_loop(0, VECS, vec_body, 0, unroll=8)
            pltpu.sync_copy(x_v, out_hbm.at[pl.ds(xbase, CHW)])
            return carry

        return lax.fori_loop(0, B, batch_body, carry)

    lax.fori_loop(0, CHUNKS, chunk_body, 0)


_pos_add = functools.partial(
    pl.kernel,
    out_type=jax.ShapeDtypeStruct((B * M * D,), jnp.float32),
    mesh=plsc.VectorSubcoreMesh(core_axis_name="c", subcore_axis_name="s"),
    scratch_types=[
        pltpu.VMEM((CHW,), jnp.float32),
        pltpu.VMEM((CHW,), jnp.float32),
    ],
)(_pos_add_body)


@jax.jit
def kernel(x, pos_table):
    out = _pos_add(x.reshape(-1), pos_table.reshape(-1))
    return out.reshape(x.shape)


# speedup vs baseline: 1.1985x; 1.1985x over previous
import functools

import jax
import jax.numpy as jnp
from jax import lax
from jax.experimental import pallas as pl
from jax.experimental.pallas import tpu as pltpu
from jax.experimental.pallas import tpu_sc as plsc

B = 4
M = 8192
D = 768
NC = 2
NS = 16
NW = NC * NS
POS_PER_W = M // NW
CH = 32
CHUNKS = POS_PER_W // CH
CHW = CH * D
VECS = CHW // 16
NIT = CHUNKS * B


def _pos_add_body(x_hbm, pos_hbm, out_hbm,
                  xv0, xv1, pv0, pv1,
                  sin0, sin1, sout0, sout1, sp0, sp1):
    wid = lax.axis_index("s") * NC + lax.axis_index("c")
    pos_base = wid * (POS_PER_W * D)
    xv = [xv0, xv1]
    pv = [pv0, pv1]
    sin = [sin0, sin1]
    sout = [sout0, sout1]
    sp = [sp0, sp1]

    def x_off(g):
        c, b = divmod(g, B)
        return b * (M * D) + pos_base + c * CHW

    in_h = [None] * NIT
    out_h = [None] * NIT
    pos_h = [None] * CHUNKS

    pos_h[0] = pltpu.async_copy(pos_hbm.at[pl.ds(pos_base, CHW)], pv[0], sp[0])
    in_h[0] = pltpu.async_copy(x_hbm.at[pl.ds(x_off(0), CHW)], xv[0], sin[0])

    for g in range(NIT):
        s = g % 2
        c = g // B
        if g % B == 0 and c + 1 < CHUNKS:
            pos_h[c + 1] = pltpu.async_copy(
                pos_hbm.at[pl.ds(pos_base + (c + 1) * CHW, CHW)],
                pv[(c + 1) % 2], sp[(c + 1) % 2])
        if g + 1 < NIT:
            if g >= 1:
                out_h[g - 1].wait()
            in_h[g + 1] = pltpu.async_copy(
                x_hbm.at[pl.ds(x_off(g + 1), CHW)], xv[1 - s], sin[1 - s])
        if g % B == 0:
            pos_h[c].wait()
        in_h[g].wait()

        pvs = pv[c % 2]
        xvs = xv[s]

        def vec_body(k, carry):
            sl = pl.ds(k * 16, 16)
            xvs[sl] += pvs[sl]
            return carry

        lax.fori_loop(0, VECS, vec_body, 0, unroll=8)
        out_h[g] = pltpu.async_copy(xvs, out_hbm.at[pl.ds(x_off(g), CHW)], sout[s])

    out_h[NIT - 2].wait()
    out_h[NIT - 1].wait()


_pos_add = functools.partial(
    pl.kernel,
    out_type=jax.ShapeDtypeStruct((B * M * D,), jnp.float32),
    mesh=plsc.VectorSubcoreMesh(core_axis_name="c", subcore_axis_name="s"),
    scratch_types=[
        pltpu.VMEM((CHW,), jnp.float32),
        pltpu.VMEM((CHW,), jnp.float32),
        pltpu.VMEM((CHW,), jnp.float32),
        pltpu.VMEM((CHW,), jnp.float32),
        pltpu.SemaphoreType.DMA,
        pltpu.SemaphoreType.DMA,
        pltpu.SemaphoreType.DMA,
        pltpu.SemaphoreType.DMA,
        pltpu.SemaphoreType.DMA,
        pltpu.SemaphoreType.DMA,
    ],
)(_pos_add_body)


@jax.jit
def kernel(x, pos_table):
    out = _pos_add(x.reshape(-1), pos_table.reshape(-1))
    return out.reshape(x.shape)


# speedup vs baseline: 1.7894x; 1.4930x over previous
import functools

import jax
import jax.numpy as jnp
from jax import lax
from jax.experimental import pallas as pl
from jax.experimental.pallas import tpu as pltpu
from jax.experimental.pallas import tpu_sc as plsc

B = 4
M = 8192
D = 768
NC = 2
NS = 16
NW = NC * NS
POS_PER_W = M // NW
CH = 32
CHUNKS = POS_PER_W // CH
CHW = CH * D
VECS = CHW // 16
NIT = CHUNKS * B


def _pos_add_body(x_hbm, pos_hbm, out_hbm,
                  xv0, xv1, pv0, pv1,
                  sin0, sin1, sout0, sout1, sp0, sp1):
    wid = lax.axis_index("s") * NC + lax.axis_index("c")
    pos_base = wid * (POS_PER_W * D)
    xv = [xv0, xv1]
    pv = [pv0, pv1]
    sin = [sin0, sin1]
    sout = [sout0, sout1]
    sp = [sp0, sp1]

    def x_off(g):
        c, b = divmod(g, B)
        return b * (M * D) + pos_base + c * CHW

    in_h = [None] * NIT
    out_h = [None] * NIT
    pos_h = [None] * CHUNKS

    pos_h[0] = pltpu.async_copy(pos_hbm.at[pl.ds(pos_base, CHW)], pv[0], sp[0])
    in_h[0] = pltpu.async_copy(x_hbm.at[pl.ds(x_off(0), CHW)], xv[0], sin[0])

    for g in range(NIT):
        s = g % 2
        c = g // B
        if g % B == 0 and c + 1 < CHUNKS:
            pos_h[c + 1] = pltpu.async_copy(
                pos_hbm.at[pl.ds(pos_base + (c + 1) * CHW, CHW)],
                pv[(c + 1) % 2], sp[(c + 1) % 2])
        if g + 1 < NIT:
            if g >= 1:
                out_h[g - 1].wait()
            in_h[g + 1] = pltpu.async_copy(
                x_hbm.at[pl.ds(x_off(g + 1), CHW)], xv[1 - s], sin[1 - s])
        if g % B == 0:
            pos_h[c].wait()
        in_h[g].wait()

        pvs = pv[c % 2]
        xvs = xv[s]

        @plsc.parallel_loop(0, VECS, unroll=16)
        def _vec_body(k):
            sl = pl.ds(k * 16, 16)
            plsc.addupdate(xvs.at[sl], pvs[sl])
        out_h[g] = pltpu.async_copy(xvs, out_hbm.at[pl.ds(x_off(g), CHW)], sout[s])

    out_h[NIT - 2].wait()
    out_h[NIT - 1].wait()


_pos_add = functools.partial(
    pl.kernel,
    out_type=jax.ShapeDtypeStruct((B * M * D,), jnp.float32),
    mesh=plsc.VectorSubcoreMesh(core_axis_name="c", subcore_axis_name="s"),
    scratch_types=[
        pltpu.VMEM((CHW,), jnp.float32),
        pltpu.VMEM((CHW,), jnp.float32),
        pltpu.VMEM((CHW,), jnp.float32),
        pltpu.VMEM((CHW,), jnp.float32),
        pltpu.SemaphoreType.DMA,
        pltpu.SemaphoreType.DMA,
        pltpu.SemaphoreType.DMA,
        pltpu.SemaphoreType.DMA,
        pltpu.SemaphoreType.DMA,
        pltpu.SemaphoreType.DMA,
    ],
)(_pos_add_body)


@jax.jit
def kernel(x, pos_table):
    out = _pos_add(x.reshape(-1), pos_table.reshape(-1))
    return out.reshape(x.shape)


# speedup vs baseline: 3.1450x; 1.7576x over previous
import functools

import jax
import jax.numpy as jnp
from jax import lax
from jax.experimental import pallas as pl
from jax.experimental.pallas import tpu as pltpu
from jax.experimental.pallas import tpu_sc as plsc

B = 4
M = 8192
D = 768
NC = 2
NS = 16
NW = NC * NS
POS_PER_W = M // NW
CH = 32
CHUNKS = POS_PER_W // CH
VPR = D // 16
VECS = CH * VPR
NIT = CHUNKS * B


def _pos_add_body(x_hbm, pos_hbm, out_hbm,
                  xv0, xv1, pv0, pv1,
                  sin0, sin1, sout0, sout1, sp0, sp1):
    wid = lax.axis_index("s") * NC + lax.axis_index("c")
    row0 = wid * POS_PER_W
    xv = [xv0, xv1]
    pv = [pv0, pv1]
    sin = [sin0, sin1]
    sout = [sout0, sout1]
    sp = [sp0, sp1]

    def x_row(g):
        c, b = divmod(g, B)
        return b * M + row0 + c * CH

    in_h = [None] * NIT
    out_h = [None] * NIT
    pos_h = [None] * CHUNKS

    pos_h[0] = pltpu.async_copy(pos_hbm.at[pl.ds(row0, CH)], pv[0], sp[0])
    in_h[0] = pltpu.async_copy(x_hbm.at[pl.ds(x_row(0), CH)], xv[0], sin[0])

    for g in range(NIT):
        s = g % 2
        c = g // B
        if g % B == 0 and c + 1 < CHUNKS:
            pos_h[c + 1] = pltpu.async_copy(
                pos_hbm.at[pl.ds(row0 + (c + 1) * CH, CH)],
                pv[(c + 1) % 2], sp[(c + 1) % 2])
        if g + 1 < NIT:
            if g >= 1:
                out_h[g - 1].wait()
            in_h[g + 1] = pltpu.async_copy(
                x_hbm.at[pl.ds(x_row(g + 1), CH)], xv[1 - s], sin[1 - s])
        if g % B == 0:
            pos_h[c].wait()
        in_h[g].wait()

        pvs = pv[c % 2]
        xvs = xv[s]

        @plsc.parallel_loop(0, VECS, unroll=16)
        def _vec_body(k):
            r = k // VPR
            j = (k % VPR) * 16
            plsc.addupdate(xvs.at[r, pl.ds(j, 16)], pvs[r, pl.ds(j, 16)])

        out_h[g] = pltpu.async_copy(xvs, out_hbm.at[pl.ds(x_row(g), CH)], sout[s])

    out_h[NIT - 2].wait()
    out_h[NIT - 1].wait()


_pos_add = functools.partial(
    pl.kernel,
    out_type=jax.ShapeDtypeStruct((B * M, D), jnp.float32),
    mesh=plsc.VectorSubcoreMesh(core_axis_name="c", subcore_axis_name="s"),
    scratch_types=[
        pltpu.VMEM((CH, D), jnp.float32),
        pltpu.VMEM((CH, D), jnp.float32),
        pltpu.VMEM((CH, D), jnp.float32),
        pltpu.VMEM((CH, D), jnp.float32),
        pltpu.SemaphoreType.DMA,
        pltpu.SemaphoreType.DMA,
        pltpu.SemaphoreType.DMA,
        pltpu.SemaphoreType.DMA,
        pltpu.SemaphoreType.DMA,
        pltpu.SemaphoreType.DMA,
    ],
)(_pos_add_body)


@jax.jit
def kernel(x, pos_table):
    out = _pos_add(x.reshape(B * M, D), pos_table)
    return out.reshape(x.shape)


# speedup vs baseline: 4.7655x; 1.5152x over previous
import functools

import jax
import jax.numpy as jnp
from jax import lax
from jax.experimental import pallas as pl
from jax.experimental.pallas import tpu as pltpu
from jax.experimental.pallas import tpu_sc as plsc

B = 4
M = 8192
D = 768
NC = 2
NS = 16
NW = NC * NS
POS_PER_W = M // NW
CH = 32
CHUNKS = POS_PER_W // CH
VPR = D // 16
VECS = CH * VPR
NIT = CHUNKS * B


def _pos_add_body(x_hbm, pos_hbm, out_hbm,
                  xv0, xv1, pv0, pv1,
                  sin0, sin1, sout0, sout1, sp0, sp1):
    wid = lax.axis_index("s") * NC + lax.axis_index("c")
    row0 = wid * POS_PER_W
    xv = [xv0, xv1]
    pv = [pv0, pv1]
    sin = [sin0, sin1]
    sout = [sout0, sout1]
    sp = [sp0, sp1]

    def x_row(g):
        c, b = divmod(g, B)
        return b * M + row0 + c * CH

    in_h = [None] * NIT
    out_h = [None] * NIT
    pos_h = [None] * CHUNKS

    pos_h[0] = pltpu.async_copy(pos_hbm.at[pl.ds(row0, CH)], pv[0], sp[0])
    in_h[0] = pltpu.async_copy(x_hbm.at[pl.ds(x_row(0), CH)], xv[0], sin[0])

    for g in range(NIT):
        s = g % 2
        c = g // B
        if g % B == 0 and c + 1 < CHUNKS:
            pos_h[c + 1] = pltpu.async_copy(
                pos_hbm.at[pl.ds(row0 + (c + 1) * CH, CH)],
                pv[(c + 1) % 2], sp[(c + 1) % 2])
        if g + 1 < NIT:
            if g >= 1:
                out_h[g - 1].wait()
            in_h[g + 1] = pltpu.async_copy(
                x_hbm.at[pl.ds(x_row(g + 1), CH)], xv[1 - s], sin[1 - s])
        if g % B == 0:
            pos_h[c].wait()
        in_h[g].wait()

        pvs = pv[c % 2]
        xvs = xv[s]

        @plsc.parallel_loop(0, CH, unroll=1)
        def _row_body(r):
            @plsc.parallel_loop(0, VPR, unroll=16)
            def _vec_body(v):
                j = v * 16
                plsc.addupdate(xvs.at[r, pl.ds(j, 16)], pvs[r, pl.ds(j, 16)])

        out_h[g] = pltpu.async_copy(xvs, out_hbm.at[pl.ds(x_row(g), CH)], sout[s])

    out_h[NIT - 2].wait()
    out_h[NIT - 1].wait()


_pos_add = functools.partial(
    pl.kernel,
    out_type=jax.ShapeDtypeStruct((B * M, D), jnp.float32),
    mesh=plsc.VectorSubcoreMesh(core_axis_name="c", subcore_axis_name="s"),
    scratch_types=[
        pltpu.VMEM((CH, D), jnp.float32),
        pltpu.VMEM((CH, D), jnp.float32),
        pltpu.VMEM((CH, D), jnp.float32),
        pltpu.VMEM((CH, D), jnp.float32),
        pltpu.SemaphoreType.DMA,
        pltpu.SemaphoreType.DMA,
        pltpu.SemaphoreType.DMA,
        pltpu.SemaphoreType.DMA,
        pltpu.SemaphoreType.DMA,
        pltpu.SemaphoreType.DMA,
    ],
)(_pos_add_body)


@jax.jit
def kernel(x, pos_table):
    out = _pos_add(x.reshape(B * M, D), pos_table)
    return out.reshape(x.shape)
